# baseline (speedup 1.0000x reference)
import functools

import jax
import jax.numpy as jnp
from jax import lax
from jax.experimental import pallas as pl
from jax.experimental.pallas import tpu as pltpu
from jax.experimental.pallas import tpu_sc as plsc

B = 16384
D = 16
V = 1000000
NC = 2
NS = 16
NW = NC * NS
RPW = B // NW
HALF = RPW // 2


def _sc_body(i_hbm, j_hbm, tbl, out, pts_i, pts_j, gi, gj, st_i, st_j,
             q_v, sem):
    c = lax.axis_index("c")
    s = lax.axis_index("s")
    wid = s * NC + c

    pltpu.sync_copy(i_hbm.at[pl.ds(wid * RPW, RPW)], pts_i)
    pltpu.sync_copy(j_hbm.at[pl.ds(wid * RPW, RPW)], pts_j)

    lane = lax.iota(jnp.int32, 16)

    def shift_blk(b, _):
        r = b * 16 + lane
        plsc.store_scatter(gi, [r], plsc.load_gather(pts_i, [r]) >> 3)
        plsc.store_scatter(gj, [r], plsc.load_gather(pts_j, [r]) >> 3)
        return _

    lax.fori_loop(0, RPW // 16, shift_blk, None)

    for half in range(2):
        copies = []
        for g, st in ((gi, st_i), (gj, st_j)):
            for sl in range(HALF // 128):
                copies.append(pltpu.async_copy(
                    tbl.at[g.at[pl.ds(half * HALF + sl * 128, 128)]],
                    st.at[pl.ds(sl * 128, 128)], sem))
        for cp in copies:
            cp.wait()

        def blk_body(b, _, half=half):
            rloc = b * 16 + lane
            pglob = half * HALF + rloc
            pv_i = plsc.load_gather(pts_i, [pglob])
            pv_j = plsc.load_gather(pts_j, [pglob])
            ci = (pv_i & 7) * 16
            cj = (pv_j & 7) * 16
            d = jnp.zeros((16,), jnp.float32)
            for f in range(D):
                zi = plsc.load_gather(st_i, [rloc, ci + f])
                zj = plsc.load_gather(st_j, [rloc, cj + f])
                df = zi - zj + 1e-6
                d = d + df * df
            q = 1.0 / (1.0 + d)
            plsc.store_scatter(q_v, [pglob], q)
            return _

        lax.fori_loop(0, HALF // 16, blk_body, None)

    pltpu.sync_copy(q_v, out.at[pl.ds(wid * RPW, RPW)])


@jax.jit
def _sc_qij(i, j, tbl2):
    mesh = plsc.VectorSubcoreMesh(core_axis_name="c", subcore_axis_name="s")
    f = pl.kernel(
        _sc_body,
        mesh=mesh,
        compiler_params=pltpu.CompilerParams(
            needs_layout_passes=False, use_tc_tiling_on_sc=False),
        out_type=jax.ShapeDtypeStruct((B,), jnp.float32),
        scratch_types=[
            pltpu.VMEM((RPW,), jnp.int32),
            pltpu.VMEM((RPW,), jnp.int32),
            pltpu.VMEM((RPW,), jnp.int32),
            pltpu.VMEM((RPW,), jnp.int32),
            pltpu.VMEM((HALF, 128), jnp.float32),
            pltpu.VMEM((HALF, 128), jnp.float32),
            pltpu.VMEM((RPW,), jnp.float32),
            pltpu.SemaphoreType.DMA,
        ],
    )
    return f(i, j, tbl2)


TBLK = 8192


def _tc_transpose_body(in_ref, out_ref):
    out_ref[...] = in_ref[...].T


@jax.jit
def _tc_relayout(tbl_t):
    grid = (V + TBLK - 1) // TBLK
    return pl.pallas_call(
        _tc_transpose_body,
        grid=(grid,),
        in_specs=[pl.BlockSpec((D, TBLK), lambda b: (0, b))],
        out_specs=pl.BlockSpec((TBLK, D), lambda b: (b, 0)),
        out_shape=jax.ShapeDtypeStruct((V, D), jnp.float32),
    )(tbl_t)


def _tc_body(p_ref, q_ref, out_ref):
    q = q_ref[...]
    p = p_ref[...]
    s = jnp.sum(q)
    log_q = jnp.log(q / s + 1e-10)
    p_log_p = jnp.where(p > 0, p * jnp.log(jnp.where(p > 0, p, 1.0)), 0.0)
    out_ref[...] = jnp.full((1, 1), jnp.sum(p_log_p - p * log_q), jnp.float32)


@jax.jit
def _tc_loss(p2d, q2d):
    return pl.pallas_call(
        _tc_body,
        out_shape=jax.ShapeDtypeStruct((1, 1), jnp.float32),
    )(p2d, q2d)


def kernel(pij, i, j, logits_weight):
    tbl_rm = _tc_relayout(logits_weight.T)
    q = _sc_qij(i, j, tbl_rm.reshape(V // 8, 8 * D))
    loss = _tc_loss(pij.reshape(128, 128), q.reshape(128, 128))
    return loss[0, 0]


# speedup vs baseline: 1.1998x; 1.1998x over previous
import functools

import jax
import jax.numpy as jnp
from jax import lax
from jax.experimental import pallas as pl
from jax.experimental.pallas import tpu as pltpu
from jax.experimental.pallas import tpu_sc as plsc

B = 16384
D = 16
NC = 2
NS = 16
NW = NC * NS
RPW = B // NW
CH = 128
NCH = RPW // CH
ROWS2D = B // CH


def _sc_body(i2d, j2d, table, out, idx_i, idx_j, rows_i, rows_j, q_v, sem):
    c = lax.axis_index("c")
    s = lax.axis_index("s")
    wid = s * NC + c
    base = wid * NCH

    pltpu.sync_copy(i2d.at[pl.ds(base, NCH)], idx_i)
    pltpu.sync_copy(j2d.at[pl.ds(base, NCH)], idx_j)

    copies = []
    for ch in range(NCH):
        copies.append(
            pltpu.async_copy(table.at[idx_i.at[ch]], rows_i.at[pl.ds(ch * CH, CH)], sem))
        copies.append(
            pltpu.async_copy(table.at[idx_j.at[ch]], rows_j.at[pl.ds(ch * CH, CH)], sem))
    for cp in copies:
        cp.wait()

    lane = lax.iota(jnp.int32, 16)

    def blk_body(blk, _):
        rid = blk * 16 + lane
        d = jnp.zeros((16,), jnp.float32)
        for k in range(D):
            kk = jnp.full((16,), k, jnp.int32)
            zi = plsc.load_gather(rows_i, [rid, kk])
            zj = plsc.load_gather(rows_j, [rid, kk])
            df = zi - zj + 1e-6
            d = d + df * df
        q = 1.0 / (1.0 + d)
        plsc.store_scatter(q_v, [rid], q)
        return _

    lax.fori_loop(0, RPW // 16, blk_body, None)

    pltpu.sync_copy(q_v, out.at[pl.ds(wid * RPW, RPW)])


@jax.jit
def _sc_qij(i2d, j2d, table):
    mesh = plsc.VectorSubcoreMesh(core_axis_name="c", subcore_axis_name="s")
    f = pl.kernel(
        _sc_body,
        mesh=mesh,
        compiler_params=pltpu.CompilerParams(
            needs_layout_passes=False, use_tc_tiling_on_sc=False),
        out_type=jax.ShapeDtypeStruct((B,), jnp.float32),
        scratch_types=[
            pltpu.VMEM((NCH, CH), jnp.int32),
            pltpu.VMEM((NCH, CH), jnp.int32),
            pltpu.VMEM((RPW, D), jnp.float32),
            pltpu.VMEM((RPW, D), jnp.float32),
            pltpu.VMEM((RPW,), jnp.float32),
            pltpu.SemaphoreType.DMA,
        ],
    )
    return f(i2d, j2d, table)


def _tc_body(p_ref, q_ref, out_ref):
    q = q_ref[...]
    p = p_ref[...]
    s = jnp.sum(q)
    log_q = jnp.log(q / s + 1e-10)
    p_log_p = jnp.where(p > 0, p * jnp.log(jnp.where(p > 0, p, 1.0)), 0.0)
    out_ref[...] = jnp.full((1, 1), jnp.sum(p_log_p - p * log_q), jnp.float32)


@jax.jit
def _tc_loss(p2d, q2d):
    return pl.pallas_call(
        _tc_body,
        out_shape=jax.ShapeDtypeStruct((1, 1), jnp.float32),
    )(p2d, q2d)


def kernel(pij, i, j, logits_weight):
    i2d = i.reshape(ROWS2D, CH)
    j2d = j.reshape(ROWS2D, CH)
    q = _sc_qij(i2d, j2d, logits_weight)
    loss = _tc_loss(pij.reshape(ROWS2D, CH), q.reshape(ROWS2D, CH))
    return loss[0, 0]
